# baseline (speedup 1.0000x reference)
import functools

import jax
import jax.numpy as jnp
from jax import lax
from jax.experimental import pallas as pl
from jax.experimental.pallas import tpu as pltpu
from jax.experimental.pallas import tpu_sc as plsc

_E = 64
_H = 1024
_I = 1024
_B = 2048
_T = 128
_BP = _B + _T


def _router_body(x_ref, wr_ref, perm_ref, dest_ref, ps_ref, offs_ref):
    x = x_ref[...]
    wr = wr_ref[...]
    logits = jnp.dot(x, wr, preferred_element_type=jnp.float32)
    m = jnp.max(logits, axis=1, keepdims=True)
    eio = lax.broadcasted_iota(jnp.int32, (_B, _E), 1)
    idx = jnp.min(jnp.where(logits == m, eio, _E), axis=1)
    prob = jax.nn.sigmoid(m)

    onehot = (idx[:, None] == eio).astype(jnp.float32)

    nb = _B // _T
    bi = lax.broadcasted_iota(jnp.int32, (_T, _T), 0)
    bj = lax.broadcasted_iota(jnp.int32, (_T, _T), 1)
    tril = (bj < bi).astype(jnp.float32)

    running = jnp.zeros((1, _E), jnp.float32)
    rank_blocks = []
    for b in range(nb):
        oh_b = onehot[b * _T:(b + 1) * _T, :]
        within = jnp.dot(tril, oh_b, preferred_element_type=jnp.float32)
        rank_b = jnp.sum((within + running) * oh_b, axis=1, keepdims=True)
        rank_blocks.append(rank_b)
        running = running + jnp.sum(oh_b, axis=0, keepdims=True)
    rank = jnp.concatenate(rank_blocks, axis=0)[:, 0]

    counts = running[0]
    ci = lax.broadcasted_iota(jnp.int32, (_E, _E), 0)
    cj = lax.broadcasted_iota(jnp.int32, (_E, _E), 1)
    upper = (ci < cj).astype(jnp.float32)
    offs_ex = jnp.dot(counts[None, :], upper,
                      preferred_element_type=jnp.float32)

    dest_f = rank + jnp.sum(onehot * offs_ex, axis=1)
    dest = dest_f.astype(jnp.int32)

    dest_row = jnp.transpose(dest[:, None])
    sio = lax.broadcasted_iota(jnp.int32, (_B, _B), 0)
    pt = (sio == dest_row).astype(jnp.float32)
    tok = lax.broadcasted_iota(jnp.int32, (_B, 1), 0).astype(jnp.float32)
    perm_col = jnp.dot(pt, tok, preferred_element_type=jnp.float32)
    ps_col = jnp.dot(pt, prob, preferred_element_type=jnp.float32)

    perm_ref[...] = perm_col.astype(jnp.int32)
    dest_ref[...] = dest[:, None]
    ps_ref[...] = ps_col
    total = jnp.full((1, 1), float(_B), jnp.float32)
    offs_ref[...] = jnp.concatenate([offs_ex, total], axis=1).astype(jnp.int32)


@jax.jit
def _router(x, wr):
    return pl.pallas_call(
        _router_body,
        out_shape=[
            jax.ShapeDtypeStruct((_B, 1), jnp.int32),
            jax.ShapeDtypeStruct((_B, 1), jnp.int32),
            jax.ShapeDtypeStruct((_B, 1), jnp.float32),
            jax.ShapeDtypeStruct((1, _E + 1), jnp.int32),
        ],
    )(x, wr)


@functools.lru_cache(maxsize=None)
def _make_sc_gather(n_idx, n_out_rows, d):
    info = plsc.get_sparse_core_info()
    nw = info.num_cores * info.num_subcores
    bpw = n_idx // nw
    mesh = plsc.VectorSubcoreMesh(core_axis_name="c", subcore_axis_name="s")

    half = bpw // 2

    @functools.partial(
        pl.kernel,
        mesh=mesh,
        out_type=jax.ShapeDtypeStruct((n_out_rows, d), jnp.float32),
        scratch_types=[
            pltpu.VMEM((half,), jnp.int32),
            pltpu.VMEM((half,), jnp.int32),
            pltpu.VMEM((half, d), jnp.float32),
            pltpu.VMEM((half, d), jnp.float32),
            pltpu.SemaphoreType.DMA,
            pltpu.SemaphoreType.DMA,
        ],
    )
    def gather_k(table_hbm, idx_hbm, out_hbm, idx_a, idx_b, rows_a, rows_b,
                 sem_a, sem_b):
        wid = lax.axis_index("s") * info.num_cores + lax.axis_index("c")
        base = wid * bpw
        pltpu.sync_copy(idx_hbm.at[pl.ds(base, half)], idx_a)
        pltpu.sync_copy(idx_hbm.at[pl.ds(base + half, half)], idx_b)
        g0 = pltpu.async_copy(table_hbm.at[idx_a], rows_a, sem_a)
        g1 = pltpu.async_copy(table_hbm.at[idx_b], rows_b, sem_b)
        g0.wait()
        pltpu.sync_copy(rows_a, out_hbm.at[pl.ds(base, half)])
        g1.wait()
        pltpu.sync_copy(rows_b, out_hbm.at[pl.ds(base + half, half)])

    return gather_k


_EPS = 1


def _ffn_body(offs_ref, xs_ref, ps_ref, wg_ref, wu_ref, wd_ref, out_ref):
    def one_expert(e, wg, wu, wd):
        off = offs_ref[e]
        end = offs_ref[e + 1]
        start0 = (off // 8) * 8

        def tile(i, _):
            start = pl.multiple_of(start0 + i * _T, 8)
            rows = xs_ref[pl.ds(start, _T), :]
            g = jnp.dot(rows, wg, preferred_element_type=jnp.float32)
            u = jnp.dot(rows, wu, preferred_element_type=jnp.float32)
            h = g * jax.nn.sigmoid(g) * u
            o = jnp.dot(h, wd, preferred_element_type=jnp.float32)
            o = o * ps_ref[pl.ds(start, _T), :]
            grow = lax.broadcasted_iota(jnp.int32, (_T, 1), 0) + start
            valid = (grow >= off) & (grow < end)
            cur = out_ref[pl.ds(start, _T), :]
            out_ref[pl.ds(start, _T), :] = jnp.where(valid, o, cur)
            return 0

        nt = (end - start0 + _T - 1) // _T
        lax.fori_loop(0, nt, tile, 0)

    base = pl.program_id(0) * _EPS
    for k in range(_EPS):
        one_expert(base + k, wg_ref[k], wu_ref[k], wd_ref[k])


@jax.jit
def _grouped_ffn(offs, xs, ps, wg, wu, wd):
    return pl.pallas_call(
        _ffn_body,
        grid=(_E // _EPS,),
        in_specs=[
            pl.BlockSpec(memory_space=pltpu.SMEM),
            pl.BlockSpec((_BP, _H), lambda e: (0, 0)),
            pl.BlockSpec((_BP, 1), lambda e: (0, 0)),
            pl.BlockSpec((_EPS, _H, _I), lambda e: (e, 0, 0)),
            pl.BlockSpec((_EPS, _H, _I), lambda e: (e, 0, 0)),
            pl.BlockSpec((_EPS, _I, _H), lambda e: (e, 0, 0)),
        ],
        out_specs=pl.BlockSpec((_BP, _H), lambda e: (0, 0)),
        out_shape=jax.ShapeDtypeStruct((_BP, _H), jnp.float32),
    )(offs, xs, ps, wg, wu, wd)


def kernel(x, Wr, Wg, Wu, Wd):
    perm2, dest2, ps2, offs2 = _router(x, Wr)
    perm = perm2.reshape(_B)
    dest = dest2.reshape(_B)
    offs = offs2.reshape(_E + 1)
    ps = jnp.concatenate([ps2, jnp.zeros((_T, 1), jnp.float32)], axis=0)

    xs = _make_sc_gather(_B, _BP, _H)(x, perm)
    osorted = _grouped_ffn(offs, xs, ps, Wg, Wu, Wd)
    return _make_sc_gather(_B, _B, _H)(osorted, dest)


# speedup vs baseline: 1.0089x; 1.0089x over previous
import functools

import jax
import jax.numpy as jnp
from jax import lax
from jax.experimental import pallas as pl
from jax.experimental.pallas import tpu as pltpu
from jax.experimental.pallas import tpu_sc as plsc

_E = 64
_H = 1024
_I = 1024
_B = 2048
_T = 128
_BP = _B + _T


def _router_body(x_ref, wr_ref, perm_ref, dest_ref, ps_ref, offs_ref):
    x = x_ref[...]
    wr = wr_ref[...]
    logits = jnp.dot(x, wr, preferred_element_type=jnp.float32)
    m = jnp.max(logits, axis=1, keepdims=True)
    eio = lax.broadcasted_iota(jnp.int32, (_B, _E), 1)
    idx = jnp.min(jnp.where(logits == m, eio, _E), axis=1)
    prob = jax.nn.sigmoid(m)

    onehot = (idx[:, None] == eio).astype(jnp.float32)

    nb = _B // _T
    bi = lax.broadcasted_iota(jnp.int32, (_T, _T), 0)
    bj = lax.broadcasted_iota(jnp.int32, (_T, _T), 1)
    tril = (bj < bi).astype(jnp.float32)

    running = jnp.zeros((1, _E), jnp.float32)
    rank_blocks = []
    for b in range(nb):
        oh_b = onehot[b * _T:(b + 1) * _T, :]
        within = jnp.dot(tril, oh_b, preferred_element_type=jnp.float32)
        rank_b = jnp.sum((within + running) * oh_b, axis=1, keepdims=True)
        rank_blocks.append(rank_b)
        running = running + jnp.sum(oh_b, axis=0, keepdims=True)
    rank = jnp.concatenate(rank_blocks, axis=0)[:, 0]

    counts = running[0]
    ci = lax.broadcasted_iota(jnp.int32, (_E, _E), 0)
    cj = lax.broadcasted_iota(jnp.int32, (_E, _E), 1)
    upper = (ci < cj).astype(jnp.float32)
    offs_ex = jnp.dot(counts[None, :], upper,
                      preferred_element_type=jnp.float32)

    dest_f = rank + jnp.sum(onehot * offs_ex, axis=1)
    dest = dest_f.astype(jnp.int32)

    dest_row = jnp.transpose(dest[:, None])
    sio = lax.broadcasted_iota(jnp.int32, (_B, _B), 0)
    pt = (sio == dest_row).astype(jnp.float32)
    tok = lax.broadcasted_iota(jnp.int32, (_B, 1), 0).astype(jnp.float32)
    perm_col = jnp.dot(pt, tok, preferred_element_type=jnp.float32)
    ps_col = jnp.dot(pt, prob, preferred_element_type=jnp.float32)

    perm_ref[...] = perm_col.astype(jnp.int32)
    dest_ref[...] = dest[:, None]
    ps_ref[...] = ps_col
    total = jnp.full((1, 1), float(_B), jnp.float32)
    offs_ref[...] = jnp.concatenate([offs_ex, total], axis=1).astype(jnp.int32)


@jax.jit
def _router(x, wr):
    return pl.pallas_call(
        _router_body,
        out_shape=[
            jax.ShapeDtypeStruct((_B, 1), jnp.int32),
            jax.ShapeDtypeStruct((_B, 1), jnp.int32),
            jax.ShapeDtypeStruct((_B, 1), jnp.float32),
            jax.ShapeDtypeStruct((1, _E + 1), jnp.int32),
        ],
    )(x, wr)


@functools.lru_cache(maxsize=None)
def _make_sc_gather(n_idx, n_out_rows, d):
    info = plsc.get_sparse_core_info()
    nw = info.num_cores * info.num_subcores
    bpw = n_idx // nw
    mesh = plsc.VectorSubcoreMesh(core_axis_name="c", subcore_axis_name="s")

    @functools.partial(
        pl.kernel,
        mesh=mesh,
        out_type=jax.ShapeDtypeStruct((n_out_rows, d), jnp.float32),
        scratch_types=[
            pltpu.VMEM((bpw,), jnp.int32),
            pltpu.VMEM((bpw, d), jnp.float32),
            pltpu.SemaphoreType.DMA,
        ],
    )
    def gather_k(table_hbm, idx_hbm, out_hbm, idx_v, rows_v, sem):
        wid = lax.axis_index("s") * info.num_cores + lax.axis_index("c")
        base = wid * bpw
        pltpu.sync_copy(idx_hbm.at[pl.ds(base, bpw)], idx_v)
        pltpu.async_copy(table_hbm.at[idx_v], rows_v, sem).wait()
        pltpu.sync_copy(rows_v, out_hbm.at[pl.ds(base, bpw)])

    return gather_k


_EPS = 1


def _ffn_body(offs_ref, xs_ref, ps_ref, wg_ref, wu_ref, wd_ref, out_ref):
    def one_expert(e, wg, wu, wd):
        off = offs_ref[e]
        end = offs_ref[e + 1]
        start0 = (off // 8) * 8

        def tile(i, _):
            start = pl.multiple_of(start0 + i * _T, 8)
            rows = xs_ref[pl.ds(start, _T), :]
            g = jnp.dot(rows, wg, preferred_element_type=jnp.float32)
            u = jnp.dot(rows, wu, preferred_element_type=jnp.float32)
            h = g * jax.nn.sigmoid(g) * u
            o = jnp.dot(h, wd, preferred_element_type=jnp.float32)
            o = o * ps_ref[pl.ds(start, _T), :]
            grow = lax.broadcasted_iota(jnp.int32, (_T, 1), 0) + start
            valid = (grow >= off) & (grow < end)
            cur = out_ref[pl.ds(start, _T), :]
            out_ref[pl.ds(start, _T), :] = jnp.where(valid, o, cur)
            return 0

        nt = (end - start0 + _T - 1) // _T
        lax.fori_loop(0, nt, tile, 0)

    base = pl.program_id(0) * _EPS
    for k in range(_EPS):
        one_expert(base + k, wg_ref[k], wu_ref[k], wd_ref[k])


@jax.jit
def _grouped_ffn(offs, xs, ps, wg, wu, wd):
    return pl.pallas_call(
        _ffn_body,
        grid=(_E // _EPS,),
        in_specs=[
            pl.BlockSpec(memory_space=pltpu.SMEM),
            pl.BlockSpec((_BP, _H), lambda e: (0, 0)),
            pl.BlockSpec((_BP, 1), lambda e: (0, 0)),
            pl.BlockSpec((_EPS, _H, _I), lambda e: (e, 0, 0)),
            pl.BlockSpec((_EPS, _H, _I), lambda e: (e, 0, 0)),
            pl.BlockSpec((_EPS, _I, _H), lambda e: (e, 0, 0)),
        ],
        out_specs=pl.BlockSpec((_BP, _H), lambda e: (0, 0)),
        out_shape=jax.ShapeDtypeStruct((_BP, _H), jnp.float32),
    )(offs, xs, ps, wg, wu, wd)


def kernel(x, Wr, Wg, Wu, Wd):
    perm2, dest2, ps2, offs2 = _router(x, Wr)
    perm = perm2.reshape(_B)
    dest = dest2.reshape(_B)
    offs = offs2.reshape(_E + 1)
    ps = jnp.concatenate([ps2, jnp.zeros((_T, 1), jnp.float32)], axis=0)

    xs = _make_sc_gather(_B, _BP, _H)(x, perm)
    osorted = _grouped_ffn(offs, xs, ps, Wg, Wu, Wd)
    return _make_sc_gather(_B, _B, _H)(osorted, dest)
